# baseline (speedup 1.0000x reference)
import functools

import jax
import jax.numpy as jnp
import numpy as np
from jax import lax
from jax.experimental import pallas as pl
from jax.experimental.pallas import tpu as pltpu
from jax.experimental.pallas import tpu_sc as plsc

N = 10000
E = 320000
D = 128
H = 64
G = 10
NPG = 1000
NA = 5
K = 500
PK = G * K
DUMP = N
PDUMP = PK

NC = 2
NS = 16
NW = NC * NS
EW = E // NW
C = 80
NCH = EW // C
ACC = 10240
ACCP = 5120
VT = 10016
X2R = 5008
NPAD = 10240

_SC_PARAMS = pltpu.CompilerParams(use_tc_tiling_on_sc=False,
                                  needs_layout_passes=False)
_MESH = plsc.VectorSubcoreMesh(core_axis_name="c", subcore_axis_name="s",
                               num_cores=2, num_subcores=16)
_MIN32 = np.int32(-2147483648)

_f32 = jnp.float32
_i32 = jnp.int32


def _zero_block(zb, dd):
    zv = jnp.zeros((16,), _f32)
    for r in range(16):
        for j in range(dd // 16):
            zb[r, pl.ds(j * 16, 16)] = zv


def _zero_acc(zb, acc, sid, rows_per_sub):
    def body(b, carry):
        pltpu.sync_copy(zb, acc.at[pl.ds(sid * rows_per_sub + b * 16, 16)])
        return carry
    lax.fori_loop(0, rows_per_sub // 16, body, 0)


def _copy_out(acc, out, cid, sid, rows_per_sub, acc_rows):
    base = sid * rows_per_sub
    pltpu.sync_copy(acc.at[pl.ds(base, rows_per_sub)],
                    out.at[pl.ds(cid * acc_rows + base, rows_per_sub)])


def _make_agg_plain(dd, with_deg):
    out_type = [jax.ShapeDtypeStruct((NC * ACC, dd), _f32)]
    if with_deg:
        out_type.append(jax.ShapeDtypeStruct((NC * ACC, 16), _f32))
    scratch = [
        pltpu.VMEM((C,), _i32),
        pltpu.VMEM((C,), _i32),
        pltpu.VMEM((C, dd), _f32),
        pltpu.VMEM((16, dd), _f32),
        pltpu.VMEM_SHARED((ACC, dd), _f32),
        pltpu.SemaphoreType.DMA,
    ]
    if with_deg:
        scratch += [
            pltpu.VMEM((C, 16), _f32),
            pltpu.VMEM((16, 16), _f32),
            pltpu.VMEM_SHARED((ACC, 16), _f32),
        ]

    def body(xt, srce, dste, *refs):
        if with_deg:
            (out, outd, sidx, didx, rows, zb, acc, sem, ones, zb2, accd) = refs
        else:
            (out, sidx, didx, rows, zb, acc, sem) = refs
        cid = lax.axis_index("c")
        sid = lax.axis_index("s")
        wid = sid * NC + cid
        _zero_block(zb, dd)
        _zero_acc(zb, acc, sid, ACC // NS)
        if with_deg:
            _zero_block(zb2, 16)
            _zero_acc(zb2, accd, sid, ACC // NS)
            ov = jnp.ones((16,), _f32)

            def fill(e, carry):
                ones[e, :] = ov
                return carry
            lax.fori_loop(0, C, fill, 0)
        plsc.subcore_barrier()
        ebase = wid * EW

        def chunk(i, carry):
            b = ebase + i * C
            pltpu.sync_copy(srce.at[pl.ds(b, C)], sidx)
            pltpu.sync_copy(dste.at[pl.ds(b, C)], didx)
            pltpu.async_copy(xt.at[sidx], rows, sem).wait()
            pltpu.sync_copy(rows, acc.at[didx], add=True)
            if with_deg:
                pltpu.sync_copy(ones, accd.at[didx], add=True)
            return carry
        lax.fori_loop(0, NCH, chunk, 0)
        plsc.subcore_barrier()
        _copy_out(acc, out, cid, sid, ACC // NS, ACC)
        if with_deg:
            _copy_out(accd, outd, cid, sid, ACC // NS, ACC)

    return pl.kernel(body, out_type=tuple(out_type) if with_deg else out_type[0],
                     mesh=_MESH, scratch_types=scratch,
                     compiler_params=_SC_PARAMS)


def _make_agg_func(dd):
    out_type = (jax.ShapeDtypeStruct((NC * ACC, dd), _f32),
                jax.ShapeDtypeStruct((NC * ACC, 16), _f32))
    scratch = [
        pltpu.VMEM((C,), _i32),
        pltpu.VMEM((C,), _i32),
        pltpu.VMEM((C,), _i32),
        pltpu.VMEM((C,), _i32),
        pltpu.VMEM((N,), _i32),
        pltpu.VMEM((C, dd), _f32),
        pltpu.VMEM((C, 16), _f32),
        pltpu.VMEM((C, 16), _f32),
        pltpu.VMEM((C, 16), _f32),
        pltpu.VMEM((16, dd), _f32),
        pltpu.VMEM((16, 16), _f32),
        pltpu.VMEM_SHARED((ACC, dd), _f32),
        pltpu.VMEM_SHARED((ACC, 16), _f32),
        pltpu.SemaphoreType.DMA,
        pltpu.SemaphoreType.DMA,
        pltpu.SemaphoreType.DMA,
    ]

    def body(xt, vt, rowe, cole, maskh, outx, outw,
             rbuf, cbuf, fsb, fdb, mk, xr, vs, vd, wb, zb, zb16,
             accx, accw, sem1, sem2, sem3):
        cid = lax.axis_index("c")
        sid = lax.axis_index("s")
        wid = sid * NC + cid
        pltpu.sync_copy(maskh, mk)
        _zero_block(zb, dd)
        _zero_acc(zb, accx, sid, ACC // NS)
        _zero_block(zb16, 16)
        _zero_acc(zb16, accw, sid, ACC // NS)
        plsc.subcore_barrier()
        ebase = wid * EW
        dumpv = jnp.full((16,), DUMP, _i32)
        zerov = jnp.zeros((16,), _i32)

        def chunk(i, carry):
            b = ebase + i * C
            pltpu.sync_copy(rowe.at[pl.ds(b, C)], rbuf)
            pltpu.sync_copy(cole.at[pl.ds(b, C)], cbuf)

            def lane(j, carry2):
                rv = rbuf[pl.ds(j * 16, 16)]
                cv = cbuf[pl.ds(j * 16, 16)]
                mr = plsc.load_gather(mk, [rv])
                mc = plsc.load_gather(mk, [cv])
                ok = (mr * mc) > 0
                fsb[pl.ds(j * 16, 16)] = jnp.where(ok, rv, zerov)
                fdb[pl.ds(j * 16, 16)] = jnp.where(ok, cv, dumpv)
                return carry2
            lax.fori_loop(0, C // 16, lane, 0)
            cp1 = pltpu.async_copy(xt.at[fsb], xr, sem1)
            cp2 = pltpu.async_copy(vt.at[fsb], vs, sem2)
            cp3 = pltpu.async_copy(vt.at[fdb], vd, sem3)
            cp1.wait()
            cp2.wait()
            cp3.wait()

            def edge(e, carry2):
                p = vs[e, :] * vd[e, :]
                s = jnp.sum(p)
                wb[e, :] = p / s
                return carry2
            lax.fori_loop(0, C, edge, 0)
            pltpu.sync_copy(xr, accx.at[fdb], add=True)
            pltpu.sync_copy(wb, accw.at[fdb], add=True)
            return carry
        lax.fori_loop(0, NCH, chunk, 0)
        plsc.subcore_barrier()
        _copy_out(accx, outx, cid, sid, ACC // NS, ACC)
        _copy_out(accw, outw, cid, sid, ACC // NS, ACC)

    return pl.kernel(body, out_type=out_type, mesh=_MESH, scratch_types=scratch,
                     compiler_params=_SC_PARAMS)


def _make_compact():
    NPW = NPAD // NW
    NCHN = NPW // C

    def body(xsp, tgt, x2, tb, xr, sem):
        cid = lax.axis_index("c")
        sid = lax.axis_index("s")
        wid = sid * NC + cid

        def chunk(i, carry):
            b = wid * NPW + i * C
            pltpu.sync_copy(tgt.at[pl.ds(b, C)], tb)
            pltpu.sync_copy(xsp.at[pl.ds(b, C)], xr)
            pltpu.async_copy(xr, x2.at[tb], sem).wait()
            return carry
        lax.fori_loop(0, NCHN, chunk, 0)

    return pl.kernel(
        body, out_type=jax.ShapeDtypeStruct((X2R, H), _f32), mesh=_MESH,
        scratch_types=[pltpu.VMEM((C,), _i32), pltpu.VMEM((C, H), _f32),
                       pltpu.SemaphoreType.DMA],
        compiler_params=_SC_PARAMS)


def _make_agg_remap():
    out_type = jax.ShapeDtypeStruct((NC * ACCP, H), _f32)
    scratch = [
        pltpu.VMEM((C,), _i32),
        pltpu.VMEM((C,), _i32),
        pltpu.VMEM((C,), _i32),
        pltpu.VMEM((C,), _i32),
        pltpu.VMEM((N,), _i32),
        pltpu.VMEM((C, H), _f32),
        pltpu.VMEM((16, H), _f32),
        pltpu.VMEM_SHARED((ACCP, H), _f32),
        pltpu.SemaphoreType.DMA,
    ]

    def body(x2t, srce, dste, ench, out, sbuf, dbuf, s2b, d2b, ek, xr, zb,
             acc, sem):
        cid = lax.axis_index("c")
        sid = lax.axis_index("s")
        wid = sid * NC + cid
        pltpu.sync_copy(ench, ek)
        _zero_block(zb, H)
        _zero_acc(zb, acc, sid, ACCP // NS)
        plsc.subcore_barrier()
        ebase = wid * EW
        dumpv = jnp.full((16,), PDUMP, _i32)
        zerov = jnp.zeros((16,), _i32)

        def chunk(i, carry):
            b = ebase + i * C
            pltpu.sync_copy(srce.at[pl.ds(b, C)], sbuf)
            pltpu.sync_copy(dste.at[pl.ds(b, C)], dbuf)

            def lane(j, carry2):
                sv = sbuf[pl.ds(j * 16, 16)]
                dv = dbuf[pl.ds(j * 16, 16)]
                es = plsc.load_gather(ek, [sv])
                ed = plsc.load_gather(ek, [dv])
                ok = (es >= 0) & (ed >= 0)
                s2b[pl.ds(j * 16, 16)] = jnp.where(ok, es, zerov)
                d2b[pl.ds(j * 16, 16)] = jnp.where(ok, ed, dumpv)
                return carry2
            lax.fori_loop(0, C // 16, lane, 0)
            pltpu.async_copy(x2t.at[s2b], xr, sem).wait()
            pltpu.sync_copy(xr, acc.at[d2b], add=True)
            return carry
        lax.fori_loop(0, NCH, chunk, 0)
        plsc.subcore_barrier()
        _copy_out(acc, out, cid, sid, ACCP // NS, ACCP)

    return pl.kernel(body, out_type=out_type, mesh=_MESH, scratch_types=scratch,
                     compiler_params=_SC_PARAMS)


def _dotT(a, b):
    return lax.dot_general(a, b, (((1,), (1,)), ((), ())),
                           preferred_element_type=_f32)


def _dot(a, b):
    return lax.dot_general(a, b, (((1,), (0,)), ((), ())),
                           preferred_element_type=_f32)


def _mlp(h, w1, b1, w2, b2):
    return _dot(jnp.maximum(_dot(h, w1) + b1, 0.0), w2) + b2


def _bn(y, g, b):
    mu = jnp.mean(y, axis=0, keepdims=True)
    var = jnp.mean((y - mu) * (y - mu), axis=0, keepdims=True)
    return g * (y - mu) / jnp.sqrt(var + 1e-5) + b


def _bn_masked(y, g, b, mf):
    cnt = jnp.sum(mf)
    xm = jnp.where(mf > 0.0, y, 0.0)
    mu = jnp.sum(xm, axis=0, keepdims=True) / cnt
    dev = jnp.where(mf > 0.0, y - mu, 0.0)
    var = jnp.sum(dev * dev, axis=0, keepdims=True) / cnt
    return g * (y - mu) / jnp.sqrt(var + 1e-5) + b


def _vtab(y, a_next):
    v = jnp.exp(_dotT(y, a_next))
    v16 = jnp.concatenate([v, jnp.zeros((N, 16 - NA), _f32)], axis=1)
    lane = lax.broadcasted_iota(_i32, (VT - N, 16), 1)
    pad = jnp.where(lane < NA, 1.0, 0.0)
    return v16, pad


def _t1_body(x_ref, p_ref, a0_ref, xs0_ref, v0_ref):
    x = x_ref[...]
    P = p_ref[...]
    lo = _dotT(x, P)
    m = jnp.max(lo, axis=1, keepdims=True)
    e = jnp.exp(lo - m)
    sm = e / jnp.sum(e, axis=1, keepdims=True)
    xs0_ref[...] = x + _dot(sm, P)
    v16, pad = _vtab(x, a0_ref[...])
    v0_ref[0:N, :] = v16
    v0_ref[N:VT, :] = pad


def _t2_body(xs0_ref, a0_ref, a1_ref, w1_ref, b1_ref, w2_ref, b2_ref,
             eps_ref, g_ref, b_ref, out_ref):
    xs0 = xs0_ref[...]
    agg = a0_ref[0:N, :] + a1_ref[0:N, :]
    h = (1.0 + eps_ref[...]) * xs0 + agg
    y = _mlp(h, w1_ref[...], b1_ref[...], w2_ref[...], b2_ref[...])
    xs1 = jnp.maximum(_bn(y, g_ref[...], b_ref[...]), 0.0)
    out_ref[0:N, :] = xs1
    out_ref[N:ACC, :] = jnp.zeros((ACC - N, H), _f32)


def _t3_body(xs1_ref, nb0_ref, nb1_ref, dg0_ref, dg1_ref, tgt_ref, enc_ref):
    xs1 = xs1_ref[0:N, :]
    nbr = nb0_ref[0:N, :] + nb1_ref[0:N, :]
    deg = dg0_ref[0:N, 0:1] + dg1_ref[0:N, 0:1]
    score = jnp.sum(jnp.abs(xs1 - nbr / jnp.maximum(deg, 1.0)),
                    axis=1, keepdims=True)
    bits = lax.bitcast_convert_type(score, _i32)
    skey = jnp.where(bits >= 0, bits,
                     jnp.bitwise_xor(jnp.bitwise_not(bits), _MIN32))
    gr = lax.broadcasted_iota(_i32, (G, N), 0)
    gc = lax.broadcasted_iota(_i32, (G, N), 1) // NPG
    Mg = jnp.where(gr == gc, 1.0, 0.0)
    tr = lax.broadcasted_iota(_i32, (N, G), 0) // NPG
    tc_ = lax.broadcasted_iota(_i32, (N, G), 1)
    Mgt = jnp.where(tr == tc_, 1.0, 0.0)
    t_full = jnp.full((N, 1), _MIN32)
    for i in range(31, -1, -1):
        step = _MIN32 if i == 31 else _i32(1 << i)
        cand = t_full + step
        cmp = jnp.where(skey >= cand, 1.0, 0.0)
        cnt = _dot(Mg, cmp)
        acc = jnp.where(cnt >= float(K), 1.0, 0.0)
        accf = _dot(Mgt, acc)
        t_full = jnp.where(accf > 0.5, cand, t_full)
    gt = skey > t_full
    tie = skey == t_full
    cnt_gt = _dot(Mg, jnp.where(gt, 1.0, 0.0))
    need_full = _dot(Mgt, float(K) - cnt_gt)
    ri = lax.broadcasted_iota(_i32, (NPG, NPG), 0)
    ci = lax.broadcasted_iota(_i32, (NPG, NPG), 1)
    Lrow = jnp.where(ci <= ri, 1.0, 0.0)
    for g in range(G):
        s0 = g * NPG
        gt_g = gt[s0:s0 + NPG, :]
        tie_g = tie[s0:s0 + NPG, :]
        c = _dot(Lrow, jnp.where(tie_g, 1.0, 0.0))
        keep_g = gt_g | (tie_g & (c <= need_full[s0:s0 + NPG, :]))
        rank = _dot(Lrow, jnp.where(keep_g, 1.0, 0.0)).astype(_i32)
        new_id = rank - 1 + g * K
        tgt_ref[s0:s0 + NPG, :] = jnp.where(keep_g, new_id, PDUMP)
        enc_ref[s0:s0 + NPG, :] = jnp.where(keep_g, new_id, -1)
    tgt_ref[N:NPAD, :] = jnp.full((NPAD - N, 1), PDUMP, _i32)
    enc_ref[N:NPAD, :] = jnp.full((NPAD - N, 1), -1, _i32)


def _t4_body(x2_ref, a0_ref, a1_ref, w1_ref, b1_ref, w2_ref, b2_ref,
             eps_ref, g_ref, b_ref, out_ref):
    xc = x2_ref[0:PK, :]
    agg = a0_ref[0:PK, :] + a1_ref[0:PK, :]
    h = (1.0 + eps_ref[...]) * xc + agg
    y = _mlp(h, w1_ref[...], b1_ref[...], w2_ref[...], b2_ref[...])
    out_ref[0:PK, :] = jnp.maximum(_bn(y, g_ref[...], b_ref[...]), 0.0)
    out_ref[PK:X2R, :] = jnp.zeros((X2R - PK, H), _f32)


def _t5_body(x3_ref, a0_ref, a1_ref, w1_ref, b1_ref, w2_ref, b2_ref,
             eps_ref, g_ref, b_ref, zs_ref):
    xc = x3_ref[0:PK, :]
    agg = a0_ref[0:PK, :] + a1_ref[0:PK, :]
    h = (1.0 + eps_ref[...]) * xc + agg
    y = _mlp(h, w1_ref[...], b1_ref[...], w2_ref[...], b2_ref[...])
    x4 = jnp.maximum(_bn(y, g_ref[...], b_ref[...]), 0.0)
    rows = []
    for g in range(G):
        blk = x4[g * K:(g + 1) * K, :]
        mean = jnp.sum(blk, axis=0, keepdims=True) / float(K)
        mx = jnp.max(blk, axis=0, keepdims=True)
        rows.append(jnp.concatenate([mean, mx], axis=1))
    zs_ref[...] = jnp.concatenate(rows, axis=0)


def _t67_body(nxt_relu, xf_ref, ax0_ref, ax1_ref, aw0_ref, aw1_ref, a_ref,
              an_ref, w1_ref, b1_ref, w2_ref, b2_ref, eps_ref, g_ref, b_ref,
              mf_ref, out_ref, vn_ref):
    dd = xf_ref.shape[1]
    xf = xf_ref[...]
    mf = mf_ref[...]
    apad = jnp.concatenate([a_ref[...], jnp.zeros((16 - NA, dd), _f32)],
                           axis=0)
    aggx = ax0_ref[0:N, :] + ax1_ref[0:N, :]
    aggw = aw0_ref[0:N, :] + aw1_ref[0:N, :]
    agg = aggx + _dot(aggw, apad)
    h = (1.0 + eps_ref[...]) * xf + agg
    y = _mlp(h, w1_ref[...], b1_ref[...], w2_ref[...], b2_ref[...])
    y = _bn_masked(y, g_ref[...], b_ref[...], mf)
    if nxt_relu:
        y = jnp.maximum(y, 0.0)
    out_ref[...] = y
    v16, pad = _vtab(y, an_ref[...])
    vn_ref[0:N, :] = v16
    vn_ref[N:VT, :] = pad


def _t8_body(xf_ref, ax0_ref, ax1_ref, aw0_ref, aw1_ref, a_ref,
             w1_ref, b1_ref, w2_ref, b2_ref, eps_ref, g_ref, b_ref, mf_ref,
             zs_ref, cw0_ref, cb0_ref, cw1_ref, cb1_ref, cw2_ref, cb2_ref,
             z_ref, zf_ref):
    xf = xf_ref[...]
    mf = mf_ref[...]
    apad = jnp.concatenate([a_ref[...], jnp.zeros((16 - NA, H), _f32)],
                           axis=0)
    aggx = ax0_ref[0:N, :] + ax1_ref[0:N, :]
    aggw = aw0_ref[0:N, :] + aw1_ref[0:N, :]
    agg = aggx + _dot(aggw, apad)
    h = (1.0 + eps_ref[...]) * xf + agg
    y = _mlp(h, w1_ref[...], b1_ref[...], w2_ref[...], b2_ref[...])
    xf3 = _bn_masked(y, g_ref[...], b_ref[...], mf)
    rows = []
    for g in range(G):
        blk = xf3[g * NPG:(g + 1) * NPG, :]
        mblk = mf[g * NPG:(g + 1) * NPG, :]
        cnt = jnp.sum(mblk)
        mean = (jnp.sum(jnp.where(mblk > 0.0, blk, 0.0), axis=0,
                        keepdims=True) / jnp.maximum(cnt, 1.0))
        mx = jnp.max(jnp.where(mblk > 0.0, blk, -jnp.inf), axis=0,
                     keepdims=True)
        rows.append(jnp.concatenate([mean, mx], axis=1))
    z_func = jnp.concatenate(rows, axis=0)
    zf_ref[...] = z_func
    z = jnp.concatenate([zs_ref[...], z_func], axis=1)
    z = jnp.maximum(_dot(z, cw0_ref[...]) + cb0_ref[...], 0.0)
    z = jnp.maximum(_dot(z, cw1_ref[...]) + cb1_ref[...], 0.0)
    z_ref[...] = _dot(z, cw2_ref[...]) + cb2_ref[...]


def _tc(body, out_shape):
    return pl.pallas_call(
        body, out_shape=out_shape,
        compiler_params=pltpu.CompilerParams(vmem_limit_bytes=96 * 1024 * 1024))


_agg_plain_128 = _make_agg_plain(D, False)
_agg_plain_64deg = _make_agg_plain(H, True)
_agg_func_128 = _make_agg_func(D)
_agg_func_64 = _make_agg_func(H)
_compact = _make_compact()
_agg_remap = _make_agg_remap()


def kernel(x, params, edge_index_struct, edge_index_func, batch, roi_mask):
    ss = edge_index_struct[0]
    sd = edge_index_struct[1]
    row = edge_index_func[0]
    col = edge_index_func[1]
    maski = roi_mask.astype(_i32)
    maskf = roi_mask.astype(_f32).reshape(N, 1)

    pre = params['pre']
    post0, post1 = params['post']
    f0, f1, f2 = params['func']
    A0, A1, A2 = params['ep_anchors']
    (cw0, cb0), (cw1, cb1), (cw2, cb2) = params['cls']

    def sp(p):
        return p['eps'].reshape(1, 1)

    def rp(v):
        return v.reshape(1, -1)

    xs0, v0t = _tc(_t1_body, (jax.ShapeDtypeStruct((N, D), _f32),
                              jax.ShapeDtypeStruct((VT, 16), _f32)))(
        x, params['np_P'], A0)

    agg1 = _agg_plain_128(xs0, ss, sd)

    ax0, aw0 = _agg_func_128(x, v0t, row, col, maski)

    xs1p = _tc(_t2_body, jax.ShapeDtypeStruct((ACC, H), _f32))(
        xs0, agg1[:ACC], agg1[ACC:], pre['w1'], rp(pre['b1']), pre['w2'],
        rp(pre['b2']), sp(pre), rp(pre['bn_g']), rp(pre['bn_b']))

    nbr, dgp = _agg_plain_64deg(xs1p, ss, sd)

    tgt2, enc2 = _tc(_t3_body, (jax.ShapeDtypeStruct((NPAD, 1), _i32),
                                jax.ShapeDtypeStruct((NPAD, 1), _i32)))(
        xs1p, nbr[:ACC], nbr[ACC:], dgp[:ACC], dgp[ACC:])
    tgt = tgt2.reshape(NPAD)
    enc = enc2.reshape(NPAD)[:N]

    x2 = _compact(xs1p, tgt)

    agp1 = _agg_remap(x2, ss, sd, enc)
    x3 = _tc(_t4_body, jax.ShapeDtypeStruct((X2R, H), _f32))(
        x2, agp1[:ACCP], agp1[ACCP:], post0['w1'], rp(post0['b1']),
        post0['w2'], rp(post0['b2']), sp(post0), rp(post0['bn_g']),
        rp(post0['bn_b']))

    agp2 = _agg_remap(x3, ss, sd, enc)
    z_struct = _tc(_t5_body, jax.ShapeDtypeStruct((G, 2 * H), _f32))(
        x3, agp2[:ACCP], agp2[ACCP:], post1['w1'], rp(post1['b1']),
        post1['w2'], rp(post1['b2']), sp(post1), rp(post1['bn_g']),
        rp(post1['bn_b']))

    xf1, v1t = _tc(functools.partial(_t67_body, True),
                   (jax.ShapeDtypeStruct((N, H), _f32),
                    jax.ShapeDtypeStruct((VT, 16), _f32)))(
        x, ax0[:ACC], ax0[ACC:], aw0[:ACC], aw0[ACC:], A0, A1,
        f0['w1'], rp(f0['b1']), f0['w2'], rp(f0['b2']), sp(f0),
        rp(f0['bn_g']), rp(f0['bn_b']), maskf)

    ax1, aw1 = _agg_func_64(xf1, v1t, row, col, maski)
    xf2, v2t = _tc(functools.partial(_t67_body, True),
                   (jax.ShapeDtypeStruct((N, H), _f32),
                    jax.ShapeDtypeStruct((VT, 16), _f32)))(
        xf1, ax1[:ACC], ax1[ACC:], aw1[:ACC], aw1[ACC:], A1, A2,
        f1['w1'], rp(f1['b1']), f1['w2'], rp(f1['b2']), sp(f1),
        rp(f1['bn_g']), rp(f1['bn_b']), maskf)

    ax2, aw2 = _agg_func_64(xf2, v2t, row, col, maski)
    z, z_func = _tc(_t8_body, (jax.ShapeDtypeStruct((G, 2), _f32),
                               jax.ShapeDtypeStruct((G, 2 * H), _f32)))(
        xf2, ax2[:ACC], ax2[ACC:], aw2[:ACC], aw2[ACC:], A2,
        f2['w1'], rp(f2['b1']), f2['w2'], rp(f2['b2']), sp(f2),
        rp(f2['bn_g']), rp(f2['bn_b']), maskf, z_struct,
        cw0, rp(cb0), cw1, rp(cb1), cw2, rp(cb2))

    return (z, z_struct, z_func)


# speedup vs baseline: 1.2012x; 1.2012x over previous
import functools

import jax
import jax.numpy as jnp
import numpy as np
from jax import lax
from jax.experimental import pallas as pl
from jax.experimental.pallas import tpu as pltpu
from jax.experimental.pallas import tpu_sc as plsc

N = 10000
E = 320000
D = 128
H = 64
G = 10
NPG = 1000
NA = 5
K = 500
PK = G * K
DUMP = N
PDUMP = PK

NC = 2
NS = 16
NW = NC * NS
EW = E // NW
C = 80
NCH = EW // C
ACC = 10240
ACCP = 5120
VT = 10016
X2R = 5008
NPAD = 10240

_SC_PARAMS = pltpu.CompilerParams(use_tc_tiling_on_sc=False,
                                  needs_layout_passes=False)
_MESH = plsc.VectorSubcoreMesh(core_axis_name="c", subcore_axis_name="s",
                               num_cores=2, num_subcores=16)
_MIN32 = np.int32(-2147483648)

_f32 = jnp.float32
_i32 = jnp.int32


def _zero_block(zb, dd):
    zv = jnp.zeros((16,), _f32)
    for r in range(16):
        for j in range(dd // 16):
            zb[r, pl.ds(j * 16, 16)] = zv


def _zero_acc(zb, acc, sid, rows_per_sub):
    def body(b, carry):
        pltpu.sync_copy(zb, acc.at[pl.ds(sid * rows_per_sub + b * 16, 16)])
        return carry
    lax.fori_loop(0, rows_per_sub // 16, body, 0)


def _copy_out(acc, out, cid, sid, rows_per_sub, acc_rows):
    base = sid * rows_per_sub
    pltpu.sync_copy(acc.at[pl.ds(base, rows_per_sub)],
                    out.at[pl.ds(cid * acc_rows + base, rows_per_sub)])


def _make_agg_plain(dd, with_deg, CP, NCHP):
    PAIRS = NCHP // 2
    out_type = [jax.ShapeDtypeStruct((NC * ACC, dd), _f32)]
    if with_deg:
        out_type.append(jax.ShapeDtypeStruct((NC * ACC, 16), _f32))
    scratch = [
        pltpu.VMEM((NCHP, CP), _i32),
        pltpu.VMEM((CP,), _i32),
        pltpu.VMEM((CP,), _i32),
        pltpu.VMEM((CP, dd), _f32),
        pltpu.VMEM((CP, dd), _f32),
        pltpu.VMEM((16, dd), _f32),
        pltpu.VMEM_SHARED((ACC, dd), _f32),
        pltpu.SemaphoreType.DMA,
        pltpu.SemaphoreType.DMA,
        pltpu.SemaphoreType.DMA,
        pltpu.SemaphoreType.DMA,
    ]
    if with_deg:
        scratch += [
            pltpu.VMEM((CP, 16), _f32),
            pltpu.VMEM((16, 16), _f32),
            pltpu.VMEM_SHARED((ACC, 16), _f32),
        ]

    def body(xt, srce2, dste2, *refs):
        if with_deg:
            (out, outd, sidx2, didxA, didxB, rowsA, rowsB, zb, acc,
             semA, semB, semDA, semDB, ones, zb2, accd) = refs
        else:
            (out, sidx2, didxA, didxB, rowsA, rowsB, zb, acc,
             semA, semB, semDA, semDB) = refs
        cid = lax.axis_index("c")
        sid = lax.axis_index("s")
        wid = sid * NC + cid
        _zero_block(zb, dd)
        _zero_acc(zb, acc, sid, ACC // NS)
        if with_deg:
            _zero_block(zb2, 16)
            _zero_acc(zb2, accd, sid, ACC // NS)
            ov = jnp.ones((16,), _f32)

            def fill(e, carry):
                ones[e, :] = ov
                return carry
            lax.fori_loop(0, CP, fill, 0)
        plsc.subcore_barrier()
        cb = wid * NCHP
        pltpu.sync_copy(srce2.at[pl.ds(cb, NCHP)], sidx2)

        def pair(k, carry):
            i = 2 * k
            cpA = pltpu.async_copy(xt.at[sidx2.at[i]], rowsA, semA)
            cpB = pltpu.async_copy(xt.at[sidx2.at[i + 1]], rowsB, semB)
            cdA = pltpu.async_copy(dste2.at[cb + i], didxA, semDA)
            cdB = pltpu.async_copy(dste2.at[cb + i + 1], didxB, semDB)
            cpA.wait()
            cdA.wait()
            pltpu.sync_copy(rowsA, acc.at[didxA], add=True)
            if with_deg:
                pltpu.sync_copy(ones, accd.at[didxA], add=True)
            cpB.wait()
            cdB.wait()
            pltpu.sync_copy(rowsB, acc.at[didxB], add=True)
            if with_deg:
                pltpu.sync_copy(ones, accd.at[didxB], add=True)
            return carry
        lax.fori_loop(0, PAIRS, pair, 0)
        if NCHP % 2 == 1:
            i = NCHP - 1
            cpA = pltpu.async_copy(xt.at[sidx2.at[i]], rowsA, semA)
            cdA = pltpu.async_copy(dste2.at[cb + i], didxA, semDA)
            cpA.wait()
            cdA.wait()
            pltpu.sync_copy(rowsA, acc.at[didxA], add=True)
            if with_deg:
                pltpu.sync_copy(ones, accd.at[didxA], add=True)
        plsc.subcore_barrier()
        _copy_out(acc, out, cid, sid, ACC // NS, ACC)
        if with_deg:
            _copy_out(accd, outd, cid, sid, ACC // NS, ACC)

    return pl.kernel(body, out_type=tuple(out_type) if with_deg else out_type[0],
                     mesh=_MESH, scratch_types=scratch,
                     compiler_params=_SC_PARAMS)


def _make_agg_func(dd):
    CF = 80
    NCHF = EW // CF
    PAIRS = NCHF // 2
    out_type = (jax.ShapeDtypeStruct((NC * ACC, dd), _f32),
                jax.ShapeDtypeStruct((NC * ACC, 16), _f32))
    scratch = [
        pltpu.VMEM((NCHF, CF), _i32),
        pltpu.VMEM((NCHF, CF), _i32),
        pltpu.VMEM((CF,), _i32),
        pltpu.VMEM((CF,), _i32),
        pltpu.VMEM((CF,), _i32),
        pltpu.VMEM((CF,), _i32),
        pltpu.VMEM((N,), _i32),
        pltpu.VMEM((CF, dd), _f32),
        pltpu.VMEM((CF, dd), _f32),
        pltpu.VMEM((CF, 16), _f32),
        pltpu.VMEM((CF, 16), _f32),
        pltpu.VMEM((CF, 16), _f32),
        pltpu.VMEM((CF, 16), _f32),
        pltpu.VMEM((CF, 16), _f32),
        pltpu.VMEM((CF, 16), _f32),
        pltpu.VMEM((16, dd), _f32),
        pltpu.VMEM((16, 16), _f32),
        pltpu.VMEM_SHARED((ACC, dd), _f32),
        pltpu.VMEM_SHARED((ACC, 16), _f32),
    ] + [pltpu.SemaphoreType.DMA] * 6

    def body(xt, vt, rowe2, cole2, maskh, outx, outw,
             row2v, col2v, fsA, fdA, fsB, fdB, mk, xrA, xrB,
             vsA, vdA, vsB, vdB, wbA, wbB, zb, zb16,
             accx, accw, sa1, sa2, sa3, sb1, sb2, sb3):
        cid = lax.axis_index("c")
        sid = lax.axis_index("s")
        wid = sid * NC + cid
        pltpu.sync_copy(maskh, mk)
        _zero_block(zb, dd)
        _zero_acc(zb, accx, sid, ACC // NS)
        _zero_block(zb16, 16)
        _zero_acc(zb16, accw, sid, ACC // NS)
        plsc.subcore_barrier()
        cb = wid * NCHF
        pltpu.sync_copy(rowe2.at[pl.ds(cb, NCHF)], row2v)
        pltpu.sync_copy(cole2.at[pl.ds(cb, NCHF)], col2v)
        dumpv = jnp.full((16,), DUMP, _i32)
        zerov = jnp.zeros((16,), _i32)

        def fsfd(i, fsb, fdb):
            def lane(j, c2):
                rv = row2v[i, pl.ds(j * 16, 16)]
                cv = col2v[i, pl.ds(j * 16, 16)]
                mr = plsc.load_gather(mk, [rv])
                mc = plsc.load_gather(mk, [cv])
                ok = (mr * mc) > 0
                fsb[pl.ds(j * 16, 16)] = jnp.where(ok, rv, zerov)
                fdb[pl.ds(j * 16, 16)] = jnp.where(ok, cv, dumpv)
                return c2
            lax.fori_loop(0, CF // 16, lane, 0)

        def gathers(fsb, fdb, xr, vs, vd, s1, s2, s3):
            c1 = pltpu.async_copy(xt.at[fsb], xr, s1)
            c2 = pltpu.async_copy(vt.at[fsb], vs, s2)
            c3 = pltpu.async_copy(vt.at[fdb], vd, s3)
            return c1, c2, c3

        def wscatter(fdb, xr, vs, vd, wb):
            def edge(e, c2):
                p = vs[e, :] * vd[e, :]
                wb[e, :] = p / jnp.sum(p)
                return c2
            lax.fori_loop(0, CF, edge, 0)
            pltpu.sync_copy(xr, accx.at[fdb], add=True)
            pltpu.sync_copy(wb, accw.at[fdb], add=True)

        def pair(k, carry):
            i = 2 * k
            fsfd(i, fsA, fdA)
            cA = gathers(fsA, fdA, xrA, vsA, vdA, sa1, sa2, sa3)
            fsfd(i + 1, fsB, fdB)
            cB = gathers(fsB, fdB, xrB, vsB, vdB, sb1, sb2, sb3)
            for c in cA:
                c.wait()
            wscatter(fdA, xrA, vsA, vdA, wbA)
            for c in cB:
                c.wait()
            wscatter(fdB, xrB, vsB, vdB, wbB)
            return carry
        lax.fori_loop(0, PAIRS, pair, 0)
        fsfd(NCHF - 1, fsA, fdA)
        cA = gathers(fsA, fdA, xrA, vsA, vdA, sa1, sa2, sa3)
        for c in cA:
            c.wait()
        wscatter(fdA, xrA, vsA, vdA, wbA)
        plsc.subcore_barrier()
        _copy_out(accx, outx, cid, sid, ACC // NS, ACC)
        _copy_out(accw, outw, cid, sid, ACC // NS, ACC)

    return pl.kernel(body, out_type=out_type, mesh=_MESH, scratch_types=scratch,
                     compiler_params=_SC_PARAMS)


def _make_compact():
    NPW = NPAD // NW
    NCHN = NPW // C

    def body(xsp, tgt, x2, tb, xr, sem):
        cid = lax.axis_index("c")
        sid = lax.axis_index("s")
        wid = sid * NC + cid

        def chunk(i, carry):
            b = wid * NPW + i * C
            pltpu.sync_copy(tgt.at[pl.ds(b, C)], tb)
            pltpu.sync_copy(xsp.at[pl.ds(b, C)], xr)
            pltpu.async_copy(xr, x2.at[tb], sem).wait()
            return carry
        lax.fori_loop(0, NCHN, chunk, 0)

    return pl.kernel(
        body, out_type=jax.ShapeDtypeStruct((X2R, H), _f32), mesh=_MESH,
        scratch_types=[pltpu.VMEM((C,), _i32), pltpu.VMEM((C, H), _f32),
                       pltpu.SemaphoreType.DMA],
        compiler_params=_SC_PARAMS)


def _make_agg_remap():
    CF = 80
    NCHF = EW // CF
    PAIRS = NCHF // 2
    out_type = jax.ShapeDtypeStruct((NC * ACCP, H), _f32)
    scratch = [
        pltpu.VMEM((NCHF, CF), _i32),
        pltpu.VMEM((NCHF, CF), _i32),
        pltpu.VMEM((CF,), _i32),
        pltpu.VMEM((CF,), _i32),
        pltpu.VMEM((CF,), _i32),
        pltpu.VMEM((CF,), _i32),
        pltpu.VMEM((N,), _i32),
        pltpu.VMEM((CF, H), _f32),
        pltpu.VMEM((CF, H), _f32),
        pltpu.VMEM((16, H), _f32),
        pltpu.VMEM_SHARED((ACCP, H), _f32),
        pltpu.SemaphoreType.DMA,
        pltpu.SemaphoreType.DMA,
    ]

    def body(x2t, srce2, dste2, ench, out, s2d, d2d, s2A, d2A, s2B, d2B,
             ek, xrA, xrB, zb, acc, semA, semB):
        cid = lax.axis_index("c")
        sid = lax.axis_index("s")
        wid = sid * NC + cid
        pltpu.sync_copy(ench, ek)
        _zero_block(zb, H)
        _zero_acc(zb, acc, sid, ACCP // NS)
        plsc.subcore_barrier()
        cb = wid * NCHF
        pltpu.sync_copy(srce2.at[pl.ds(cb, NCHF)], s2d)
        pltpu.sync_copy(dste2.at[pl.ds(cb, NCHF)], d2d)
        dumpv = jnp.full((16,), PDUMP, _i32)
        zerov = jnp.zeros((16,), _i32)

        def remap(i, s2b, d2b):
            def lane(j, c2):
                sv = s2d[i, pl.ds(j * 16, 16)]
                dv = d2d[i, pl.ds(j * 16, 16)]
                es = plsc.load_gather(ek, [sv])
                ed = plsc.load_gather(ek, [dv])
                ok = (es >= 0) & (ed >= 0)
                s2b[pl.ds(j * 16, 16)] = jnp.where(ok, es, zerov)
                d2b[pl.ds(j * 16, 16)] = jnp.where(ok, ed, dumpv)
                return c2
            lax.fori_loop(0, CF // 16, lane, 0)

        def pair(k, carry):
            i = 2 * k
            remap(i, s2A, d2A)
            cpA = pltpu.async_copy(x2t.at[s2A], xrA, semA)
            remap(i + 1, s2B, d2B)
            cpB = pltpu.async_copy(x2t.at[s2B], xrB, semB)
            cpA.wait()
            pltpu.sync_copy(xrA, acc.at[d2A], add=True)
            cpB.wait()
            pltpu.sync_copy(xrB, acc.at[d2B], add=True)
            return carry
        lax.fori_loop(0, PAIRS, pair, 0)
        remap(NCHF - 1, s2A, d2A)
        cpA = pltpu.async_copy(x2t.at[s2A], xrA, semA)
        cpA.wait()
        pltpu.sync_copy(xrA, acc.at[d2A], add=True)
        plsc.subcore_barrier()
        _copy_out(acc, out, cid, sid, ACCP // NS, ACCP)

    return pl.kernel(body, out_type=out_type, mesh=_MESH, scratch_types=scratch,
                     compiler_params=_SC_PARAMS)


def _dotT(a, b, hi=True):
    return lax.dot_general(a, b, (((1,), (1,)), ((), ())),
                           preferred_element_type=_f32,
                           precision=lax.Precision.HIGHEST if hi else None)


def _dot(a, b, hi=True):
    return lax.dot_general(a, b, (((1,), (0,)), ((), ())),
                           preferred_element_type=_f32,
                           precision=lax.Precision.HIGHEST if hi else None)


def _mlp(h, w1, b1, w2, b2):
    return _dot(jnp.maximum(_dot(h, w1) + b1, 0.0), w2) + b2


def _bn(y, g, b):
    mu = jnp.mean(y, axis=0, keepdims=True)
    var = jnp.mean((y - mu) * (y - mu), axis=0, keepdims=True)
    return g * (y - mu) / jnp.sqrt(var + 1e-5) + b


def _bn_masked(y, g, b, mf):
    cnt = jnp.sum(mf)
    xm = jnp.where(mf > 0.0, y, 0.0)
    mu = jnp.sum(xm, axis=0, keepdims=True) / cnt
    dev = jnp.where(mf > 0.0, y - mu, 0.0)
    var = jnp.sum(dev * dev, axis=0, keepdims=True) / cnt
    return g * (y - mu) / jnp.sqrt(var + 1e-5) + b


def _vtab(y, a_next):
    v = jnp.exp(_dotT(y, a_next))
    v16 = jnp.concatenate([v, jnp.zeros((N, 16 - NA), _f32)], axis=1)
    lane = lax.broadcasted_iota(_i32, (VT - N, 16), 1)
    pad = jnp.where(lane < NA, 1.0, 0.0)
    return v16, pad


def _t1_body(x_ref, p_ref, a0_ref, f0w1_ref, xs0_ref, xw0_ref,
             v0_ref):
    x = x_ref[...]
    P = p_ref[...]
    lo = _dotT(x, P, hi=False)
    m = jnp.max(lo, axis=1, keepdims=True)
    e = jnp.exp(lo - m)
    sm = e / jnp.sum(e, axis=1, keepdims=True)
    xs0_ref[...] = x + _dot(sm, P, hi=False)
    xw0_ref[...] = _dot(x, f0w1_ref[...])
    v16, pad = _vtab(x, a0_ref[...])
    v0_ref[0:N, :] = v16
    v0_ref[N:VT, :] = pad


def _t2_body(xs0_ref, a0_ref, a1_ref, w1_ref, b1_ref, w2_ref, b2_ref,
             eps_ref, g_ref, b_ref, p0w1_ref, out_ref, outw_ref):
    xs0 = xs0_ref[...]
    agg = a0_ref[0:N, :] + a1_ref[0:N, :]
    h = (1.0 + eps_ref[...]) * xs0 + agg
    y = _dot(jnp.maximum(_dot(h, w1_ref[...], hi=False) + b1_ref[...], 0.0),
             w2_ref[...], hi=False) + b2_ref[...]
    xs1 = jnp.maximum(_bn(y, g_ref[...], b_ref[...]), 0.0)
    out_ref[0:N, :] = xs1
    out_ref[N:ACC, :] = jnp.zeros((ACC - N, H), _f32)
    outw_ref[0:N, :] = _dot(xs1, p0w1_ref[...])
    outw_ref[N:ACC, :] = jnp.zeros((ACC - N, H), _f32)


def _t3_body(xs1_ref, nb0_ref, nb1_ref, dg0_ref, dg1_ref, tgt_ref, enc_ref):
    xs1 = xs1_ref[0:N, :]
    nbr = nb0_ref[0:N, :] + nb1_ref[0:N, :]
    deg = dg0_ref[0:N, 0:1] + dg1_ref[0:N, 0:1]
    score = jnp.sum(jnp.abs(xs1 - nbr / jnp.maximum(deg, 1.0)),
                    axis=1, keepdims=True)
    bits = lax.bitcast_convert_type(score, _i32)
    skey = jnp.where(bits >= 0, bits,
                     jnp.bitwise_xor(jnp.bitwise_not(bits), _MIN32))
    gr = lax.broadcasted_iota(_i32, (G, N), 0)
    gc = lax.broadcasted_iota(_i32, (G, N), 1) // NPG
    Mg = jnp.where(gr == gc, 1.0, 0.0)
    tr = lax.broadcasted_iota(_i32, (N, G), 0) // NPG
    tc_ = lax.broadcasted_iota(_i32, (N, G), 1)
    Mgt = jnp.where(tr == tc_, 1.0, 0.0)
    t_full = jnp.full((N, 1), _MIN32)
    for i in range(31, -1, -1):
        step = _MIN32 if i == 31 else _i32(1 << i)
        cand = t_full + step
        cmp = jnp.where(skey >= cand, 1.0, 0.0)
        cnt = _dot(Mg, cmp, hi=False)
        acc = jnp.where(cnt >= float(K), 1.0, 0.0)
        accf = _dot(Mgt, acc, hi=False)
        t_full = jnp.where(accf > 0.5, cand, t_full)
    gt = skey > t_full
    tie = skey == t_full
    cnt_gt = _dot(Mg, jnp.where(gt, 1.0, 0.0), hi=False)
    need_full = _dot(Mgt, float(K) - cnt_gt, hi=False)
    ri = lax.broadcasted_iota(_i32, (NPG, NPG), 0)
    ci = lax.broadcasted_iota(_i32, (NPG, NPG), 1)
    Lrow = jnp.where(ci <= ri, 1.0, 0.0)
    for g in range(G):
        s0 = g * NPG
        gt_g = gt[s0:s0 + NPG, :]
        tie_g = tie[s0:s0 + NPG, :]
        c = _dot(Lrow, jnp.where(tie_g, 1.0, 0.0), hi=False)
        keep_g = gt_g | (tie_g & (c <= need_full[s0:s0 + NPG, :]))
        rank = _dot(Lrow, jnp.where(keep_g, 1.0, 0.0), hi=False).astype(_i32)
        new_id = rank - 1 + g * K
        tgt_ref[s0:s0 + NPG, :] = jnp.where(keep_g, new_id, PDUMP)
        enc_ref[s0:s0 + NPG, :] = jnp.where(keep_g, new_id, -1)
    tgt_ref[N:NPAD, :] = jnp.full((NPAD - N, 1), PDUMP, _i32)
    enc_ref[N:NPAD, :] = jnp.full((NPAD - N, 1), -1, _i32)


def _t4_body(x2w_ref, a0_ref, a1_ref, b1_ref, w2_ref, b2_ref,
             eps_ref, g_ref, b_ref, p1w1_ref, out_ref):
    xcw = x2w_ref[0:PK, :]
    agg = a0_ref[0:PK, :] + a1_ref[0:PK, :]
    hw1 = (1.0 + eps_ref[...]) * xcw + agg
    y = _dot(jnp.maximum(hw1 + b1_ref[...], 0.0), w2_ref[...]) + b2_ref[...]
    x3 = jnp.maximum(_bn(y, g_ref[...], b_ref[...]), 0.0)
    out_ref[0:PK, :] = _dot(x3, p1w1_ref[...])
    out_ref[PK:X2R, :] = jnp.zeros((X2R - PK, H), _f32)


def _t5_body(x3w_ref, a0_ref, a1_ref, b1_ref, w2_ref, b2_ref,
             eps_ref, g_ref, b_ref, zs_ref):
    xcw = x3w_ref[0:PK, :]
    agg = a0_ref[0:PK, :] + a1_ref[0:PK, :]
    hw1 = (1.0 + eps_ref[...]) * xcw + agg
    y = _dot(jnp.maximum(hw1 + b1_ref[...], 0.0), w2_ref[...]) + b2_ref[...]
    x4 = jnp.maximum(_bn(y, g_ref[...], b_ref[...]), 0.0)
    rows = []
    for g in range(G):
        blk = x4[g * K:(g + 1) * K, :]
        mean = jnp.sum(blk, axis=0, keepdims=True) / float(K)
        mx = jnp.max(blk, axis=0, keepdims=True)
        rows.append(jnp.concatenate([mean, mx], axis=1))
    zs_ref[...] = jnp.concatenate(rows, axis=0)


def _t67_body(xfw_ref, ax0_ref, ax1_ref, aw0_ref, aw1_ref, a_ref, w1_ref,
              an_ref, nw1_ref, b1_ref, w2_ref, b2_ref, eps_ref, g_ref, b_ref,
              mf_ref, out_ref, vn_ref):
    da = a_ref.shape[1]
    xfw = xfw_ref[...]
    mf = mf_ref[...]
    apad = jnp.concatenate([a_ref[...], jnp.zeros((16 - NA, da), _f32)],
                           axis=0)
    aw1 = _dot(apad, w1_ref[...])
    aggx = ax0_ref[0:N, :] + ax1_ref[0:N, :]
    aggw = aw0_ref[0:N, :] + aw1_ref[0:N, :]
    hw1 = (1.0 + eps_ref[...]) * xfw + aggx + _dot(aggw, aw1)
    y = _dot(jnp.maximum(hw1 + b1_ref[...], 0.0), w2_ref[...]) + b2_ref[...]
    y = _bn_masked(y, g_ref[...], b_ref[...], mf)
    y = jnp.maximum(y, 0.0)
    out_ref[...] = _dot(y, nw1_ref[...])
    v16, pad = _vtab(y, an_ref[...])
    vn_ref[0:N, :] = v16
    vn_ref[N:VT, :] = pad


def _t8_body(xfw_ref, ax0_ref, ax1_ref, aw0_ref, aw1_ref, a_ref, w1_ref,
             b1_ref, w2_ref, b2_ref, eps_ref, g_ref, b_ref, mf_ref,
             zs_ref, cw0_ref, cb0_ref, cw1_ref, cb1_ref, cw2_ref, cb2_ref,
             z_ref, zf_ref):
    xfw = xfw_ref[...]
    mf = mf_ref[...]
    apad = jnp.concatenate([a_ref[...], jnp.zeros((16 - NA, H), _f32)],
                           axis=0)
    aw1 = _dot(apad, w1_ref[...])
    aggx = ax0_ref[0:N, :] + ax1_ref[0:N, :]
    aggw = aw0_ref[0:N, :] + aw1_ref[0:N, :]
    hw1 = (1.0 + eps_ref[...]) * xfw + aggx + _dot(aggw, aw1)
    y = _dot(jnp.maximum(hw1 + b1_ref[...], 0.0), w2_ref[...]) + b2_ref[...]
    xf3 = _bn_masked(y, g_ref[...], b_ref[...], mf)
    rows = []
    for g in range(G):
        blk = xf3[g * NPG:(g + 1) * NPG, :]
        mblk = mf[g * NPG:(g + 1) * NPG, :]
        cnt = jnp.sum(mblk)
        mean = (jnp.sum(jnp.where(mblk > 0.0, blk, 0.0), axis=0,
                        keepdims=True) / jnp.maximum(cnt, 1.0))
        mx = jnp.max(jnp.where(mblk > 0.0, blk, -jnp.inf), axis=0,
                     keepdims=True)
        rows.append(jnp.concatenate([mean, mx], axis=1))
    z_func = jnp.concatenate(rows, axis=0)
    zf_ref[...] = z_func
    z = jnp.concatenate([zs_ref[...], z_func], axis=1)
    z = jnp.maximum(_dot(z, cw0_ref[...]) + cb0_ref[...], 0.0)
    z = jnp.maximum(_dot(z, cw1_ref[...]) + cb1_ref[...], 0.0)
    z_ref[...] = _dot(z, cw2_ref[...]) + cb2_ref[...]


def _tc(body, out_shape):
    return pl.pallas_call(
        body, out_shape=out_shape,
        compiler_params=pltpu.CompilerParams(vmem_limit_bytes=96 * 1024 * 1024))


_agg_plain_128 = _make_agg_plain(D, False, 80, 125)
_agg_plain_64deg = _make_agg_plain(H, True, 128, 80)
_agg_func_64 = _make_agg_func(H)
_compact = _make_compact()
_agg_remap = _make_agg_remap()


def kernel(x, params, edge_index_struct, edge_index_func, batch, roi_mask):
    ss = edge_index_struct[0]
    sd = edge_index_struct[1]
    row = edge_index_func[0]
    col = edge_index_func[1]
    nrow = E // 125
    ss125 = jnp.concatenate(
        [ss.reshape(nrow, 125), jnp.zeros((nrow, 3), _i32)], axis=1)
    sd125 = jnp.concatenate(
        [sd.reshape(nrow, 125), jnp.full((nrow, 3), ACC - 16, _i32)], axis=1)
    ss80 = ss.reshape(E // 80, 80)
    sd80 = sd.reshape(E // 80, 80)
    row80 = row.reshape(E // 80, 80)
    col80 = col.reshape(E // 80, 80)
    maski = roi_mask.astype(_i32)
    maskf = roi_mask.astype(_f32).reshape(N, 1)

    pre = params['pre']
    post0, post1 = params['post']
    f0, f1, f2 = params['func']
    A0, A1, A2 = params['ep_anchors']
    (cw0, cb0), (cw1, cb1), (cw2, cb2) = params['cls']

    def sp(p):
        return p['eps'].reshape(1, 1)

    def rp(v):
        return v.reshape(1, -1)

    xs0, xw0, v0t = _tc(_t1_body, (jax.ShapeDtypeStruct((N, D), _f32),
                                   jax.ShapeDtypeStruct((N, H), _f32),
                                   jax.ShapeDtypeStruct((VT, 16), _f32)))(
        x, params['np_P'], A0, f0['w1'])

    agg1 = _agg_plain_128(xs0, ss80, sd80)

    ax0, aw0 = _agg_func_64(xw0, v0t, row80, col80, maski)

    xs1p, xs1wp = _tc(_t2_body, (jax.ShapeDtypeStruct((ACC, H), _f32),
                                 jax.ShapeDtypeStruct((ACC, H), _f32)))(
        xs0, agg1[:ACC], agg1[ACC:], pre['w1'], rp(pre['b1']), pre['w2'],
        rp(pre['b2']), sp(pre), rp(pre['bn_g']), rp(pre['bn_b']),
        post0['w1'])

    nbr, dgp = _agg_plain_64deg(xs1p, ss125, sd125)

    tgt2, enc2 = _tc(_t3_body, (jax.ShapeDtypeStruct((NPAD, 1), _i32),
                                jax.ShapeDtypeStruct((NPAD, 1), _i32)))(
        xs1p, nbr[:ACC], nbr[ACC:], dgp[:ACC], dgp[ACC:])
    tgt = tgt2.reshape(NPAD)
    enc = enc2.reshape(NPAD)[:N]

    x2w = _compact(xs1wp, tgt)

    agp1 = _agg_remap(x2w, ss80, sd80, enc)
    x3w = _tc(_t4_body, jax.ShapeDtypeStruct((X2R, H), _f32))(
        x2w, agp1[:ACCP], agp1[ACCP:], rp(post0['b1']),
        post0['w2'], rp(post0['b2']), sp(post0), rp(post0['bn_g']),
        rp(post0['bn_b']), post1['w1'])

    agp2 = _agg_remap(x3w, ss80, sd80, enc)
    z_struct = _tc(_t5_body, jax.ShapeDtypeStruct((G, 2 * H), _f32))(
        x3w, agp2[:ACCP], agp2[ACCP:], rp(post1['b1']),
        post1['w2'], rp(post1['b2']), sp(post1), rp(post1['bn_g']),
        rp(post1['bn_b']))

    xf1w, v1t = _tc(_t67_body,
                    (jax.ShapeDtypeStruct((N, H), _f32),
                     jax.ShapeDtypeStruct((VT, 16), _f32)))(
        xw0, ax0[:ACC], ax0[ACC:], aw0[:ACC], aw0[ACC:], A0, f0['w1'],
        A1, f1['w1'], rp(f0['b1']), f0['w2'], rp(f0['b2']), sp(f0),
        rp(f0['bn_g']), rp(f0['bn_b']), maskf)

    ax1, aw1 = _agg_func_64(xf1w, v1t, row80, col80, maski)
    xf2w, v2t = _tc(_t67_body,
                    (jax.ShapeDtypeStruct((N, H), _f32),
                     jax.ShapeDtypeStruct((VT, 16), _f32)))(
        xf1w, ax1[:ACC], ax1[ACC:], aw1[:ACC], aw1[ACC:], A1, f1['w1'],
        A2, f2['w1'], rp(f1['b1']), f1['w2'], rp(f1['b2']), sp(f1),
        rp(f1['bn_g']), rp(f1['bn_b']), maskf)

    ax2, aw2 = _agg_func_64(xf2w, v2t, row80, col80, maski)
    z, z_func = _tc(_t8_body, (jax.ShapeDtypeStruct((G, 2), _f32),
                               jax.ShapeDtypeStruct((G, 2 * H), _f32)))(
        xf2w, ax2[:ACC], ax2[ACC:], aw2[:ACC], aw2[ACC:], A2, f2['w1'],
        rp(f2['b1']), f2['w2'], rp(f2['b2']), sp(f2),
        rp(f2['bn_g']), rp(f2['bn_b']), maskf, z_struct,
        cw0, rp(cb0), cw1, rp(cb1), cw2, rp(cb2))

    return (z, z_struct, z_func)
